# baseline (speedup 1.0000x reference)
import jax
import jax.numpy as jnp
from jax.experimental import pallas as pl
from jax.experimental.pallas import tpu as pltpu

E = 16
TOPK = 2
RANK = 32
D = 2048
QO = 2048
VO = 512
N = 2048
SCALE = 32.0 / 32.0
ER = E * RANK

BN = 256


def _fused_kernel(h_ref, rw_ref, ab_ref, qb_ref, vb_ref,
                  q_out_ref, v_out_ref):
    h = h_ref[...]
    hb = h.astype(jnp.bfloat16)

    logits = jax.lax.dot_general(
        h, rw_ref[...], (((1,), (1,)), ((), ())),
        preferred_element_type=jnp.float32)
    eiota = jax.lax.broadcasted_iota(jnp.int32, logits.shape, 1)
    m1 = jnp.max(logits, axis=-1, keepdims=True)
    i1 = jnp.min(jnp.where(logits == m1, eiota, E), axis=-1, keepdims=True)
    masked = jnp.where(eiota == i1, -jnp.inf, logits)
    m2 = jnp.max(masked, axis=-1, keepdims=True)
    i2 = jnp.min(jnp.where(masked == m2, eiota, E), axis=-1, keepdims=True)
    z = jnp.exp(m2 - m1)
    denom = 1.0 + z
    s1 = (1.0 / denom) * SCALE
    s2 = (z / denom) * SCALE

    col_expert = (jax.lax.broadcasted_iota(jnp.int32, (h.shape[0], 2 * ER), 1)
                  // RANK) % E
    w_rep = jnp.where(col_expert == i1, s1, 0.0) + jnp.where(col_expert == i2, s2, 0.0)

    low = jax.lax.dot_general(
        hb, ab_ref[...], (((1,), (0,)), ((), ())),
        preferred_element_type=jnp.float32)
    low_m = (low * w_rep).astype(jnp.bfloat16)

    q_out_ref[...] = jax.lax.dot_general(
        low_m[:, :ER], qb_ref[...], (((1,), (0,)), ((), ())),
        preferred_element_type=jnp.float32)
    v_out_ref[...] = jax.lax.dot_general(
        low_m[:, ER:], vb_ref[...], (((1,), (0,)), ((), ())),
        preferred_element_type=jnp.float32)


@jax.jit
def kernel(hidden_states, router_weight, q_lora_a, q_lora_b, v_lora_a, v_lora_b):
    qa2 = q_lora_a.transpose(1, 0, 2).reshape(D, ER).astype(jnp.bfloat16)
    va2 = v_lora_a.transpose(1, 0, 2).reshape(D, ER).astype(jnp.bfloat16)
    ab2 = jnp.concatenate([qa2, va2], axis=1)
    qb2 = q_lora_b.reshape(ER, QO).astype(jnp.bfloat16)
    vb2 = v_lora_b.reshape(ER, VO).astype(jnp.bfloat16)

    grid = (N // BN,)
    q_delta, v_delta = pl.pallas_call(
        _fused_kernel,
        grid=grid,
        in_specs=[
            pl.BlockSpec((BN, D), lambda i: (i, 0)),
            pl.BlockSpec((E, D), lambda i: (0, 0)),
            pl.BlockSpec((D, 2 * ER), lambda i: (0, 0)),
            pl.BlockSpec((ER, QO), lambda i: (0, 0)),
            pl.BlockSpec((ER, VO), lambda i: (0, 0)),
        ],
        out_specs=[
            pl.BlockSpec((BN, QO), lambda i: (i, 0)),
            pl.BlockSpec((BN, VO), lambda i: (i, 0)),
        ],
        out_shape=[
            jax.ShapeDtypeStruct((N, QO), jnp.float32),
            jax.ShapeDtypeStruct((N, VO), jnp.float32),
        ],
        compiler_params=pltpu.CompilerParams(
            dimension_semantics=("arbitrary",),
        ),
    )(hidden_states, router_weight, ab2, qb2, vb2)
    return (q_delta, v_delta)
